# baseline (speedup 1.0000x reference)
import functools
import math

import jax
import jax.numpy as jnp
from jax import lax
from jax.experimental import pallas as pl
from jax.experimental.pallas import tpu as pltpu
from jax.experimental.pallas import tpu_sc as plsc

D_MODEL = 64
SCALE = math.sqrt(D_MODEL)

_NUM_CORES = 2
_NUM_SUBCORES = 16
_NW = _NUM_CORES * _NUM_SUBCORES

_SUB = 128
_CHUNK = 640
_NSUB = _CHUNK // _SUB
_LANES = 16


@functools.partial(jax.jit, static_argnames=("n",))
def _embed_flat(table, idx, *, n):
    npw = n // _NW
    nchunk = npw // _CHUNK

    mesh = plsc.VectorSubcoreMesh(core_axis_name="c", subcore_axis_name="s")

    @functools.partial(
        pl.kernel,
        out_type=jax.ShapeDtypeStruct((n, D_MODEL), jnp.float32),
        mesh=mesh,
        compiler_params=pltpu.CompilerParams(use_tc_tiling_on_sc=False),
        scratch_types=[
            pltpu.VMEM((npw,), jnp.int32),
            pltpu.VMEM((_CHUNK, D_MODEL), jnp.float32),
            pltpu.VMEM((_CHUNK, D_MODEL), jnp.float32),
            pltpu.SemaphoreType.DMA,
            pltpu.SemaphoreType.DMA,
            pltpu.SemaphoreType.DMA,
            pltpu.SemaphoreType.DMA,
        ],
    )
    def emb(table_hbm, idx_hbm, out_hbm, idx_v, rows0, rows1, g0, g1, s0, s1):
        wid = lax.axis_index("s") * _NUM_CORES + lax.axis_index("c")
        base = wid * npw

        pltpu.sync_copy(idx_hbm.at[pl.ds(base, npw)], idx_v)

        rows = (rows0, rows1)
        gsem = (g0, g1)
        ssem = (s0, s1)
        pend_gather = [None, None]
        pend_scatter = [None, None]

        def start_gather(g):
            b = g & 1
            descs = []
            for j in range(_NSUB):
                descs.append(
                    pltpu.async_copy(
                        table_hbm.at[idx_v.at[pl.ds(g * _CHUNK + j * _SUB, _SUB)]],
                        rows[b].at[pl.ds(j * _SUB, _SUB)],
                        gsem[b],
                    )
                )
            pend_gather[b] = descs

        start_gather(0)
        for g in range(nchunk):
            b = g & 1
            if g + 1 < nchunk:
                nb = (g + 1) & 1
                if pend_scatter[nb] is not None:
                    pend_scatter[nb].wait()
                    pend_scatter[nb] = None
                start_gather(g + 1)
            for d in pend_gather[b]:
                d.wait()

            buf = rows[b]

            @plsc.parallel_loop(0, _CHUNK, unroll=4)
            def _scale(i):
                for j in range(D_MODEL // _LANES):
                    sl = pl.ds(j * _LANES, _LANES)
                    buf[i, sl] = buf[i, sl] * SCALE

            pend_scatter[b] = pltpu.async_copy(
                buf, out_hbm.at[pl.ds(base + g * _CHUNK, _CHUNK)], ssem[b]
            )

        for b in range(2):
            if pend_scatter[b] is not None:
                pend_scatter[b].wait()

    return emb(table, idx)


def kernel(x, table):
    n = x.size
    idx = x.reshape(n).astype(jnp.int32)
    out = _embed_flat(table, idx, n=n)
    return out.reshape(x.shape + (D_MODEL,))
